# Initial kernel scaffold; baseline (speedup 1.0000x reference)
import jax
import jax.numpy as jnp
from jax.experimental import pallas as pl


def kernel(tweet, num_prop, cat_prop, community_embedding, edge_community_weight, W_tw, b_tw, g_tw, be_tw, W_np, b_np, g_np, be_np, W_cp, b_cp, g_cp, be_cp, W_cm, b_cm, g_cm, be_cm, W_i1, b_i1, g_i1, be_i1, W_i2, b_i2, g_i2, be_i2, Wrel1, Wroot1, brg1, g_bn1, be_bn1, Wrel2, Wroot2, brg2, g_bn2, be_bn2, W_o1, b_o1, g_o1, be_o1, W_o2, b_o2, edge_index, edge_type):
    raise NotImplementedError("write your pallas kernel here")


# speedup vs baseline: 3.3082x; 3.3082x over previous
import functools

import jax
import jax.numpy as jnp
import numpy as np
from jax import lax
from jax.experimental import pallas as pl
from jax.experimental.pallas import tpu as pltpu
from jax.experimental.pallas import tpu_sc as plsc

N = 10000
E = 320000
D = 128

NC = 2
NS = 16
CH = 128
NCHUNK = 79
EPT = NCHUNK * CH
EPC = NS * EPT
E_PAD = NC * EPC
ACC_ROWS = 10240
ROWS_PT = ACC_ROWS // NS
INVSZ = 2 * N + 480
CNT_PT = INVSZ // NS

_BN_INV = float(1.0 / np.sqrt(np.float32(1.0 + 1e-5)))

_MESH = plsc.VectorSubcoreMesh(core_axis_name="c", subcore_axis_name="s")


def _lrelu(x):
    return jnp.where(x >= 0, x, 0.01 * x)


def _bn(x, g, b):
    return x * (g * _BN_INV) + b


def _sc_count_body(dst_hbm, et_hbm, cnt_out, dstv, etv, cidxv, onesv, zerov, cacc):
    c = lax.axis_index("c")
    s = lax.axis_index("s")
    ebase = c * EPC + s * EPT

    for j in range(8):
        sl = pl.ds(j * 16, 16)
        onesv[sl] = jnp.full((16,), 1.0, jnp.float32)
    def _z(i, _):
        zerov[pl.ds(i * 16, 16)] = jnp.zeros((16,), jnp.float32)
        return 0
    lax.fori_loop(0, CNT_PT // 16, _z, 0)
    pltpu.sync_copy(zerov, cacc.at[pl.ds(s * CNT_PT, CNT_PT)])
    plsc.subcore_barrier()

    def body(g, _):
        off = ebase + g * CH
        pltpu.sync_copy(dst_hbm.at[pl.ds(off, CH)], dstv)
        pltpu.sync_copy(et_hbm.at[pl.ds(off, CH)], etv)
        for j in range(8):
            sl = pl.ds(j * 16, 16)
            cidxv[sl] = dstv[sl] * 2 + etv[sl]
        pltpu.sync_copy(onesv, cacc.at[cidxv], add=True)
        return 0
    lax.fori_loop(0, NCHUNK, body, 0)
    plsc.subcore_barrier()
    pltpu.sync_copy(cacc.at[pl.ds(s * CNT_PT, CNT_PT)],
                    cnt_out.at[pl.ds(c * INVSZ + s * CNT_PT, CNT_PT)])


_sc_count = functools.partial(
    pl.kernel,
    out_type=jax.ShapeDtypeStruct((NC * INVSZ,), jnp.float32),
    mesh=_MESH,
    compiler_params=pltpu.CompilerParams(needs_layout_passes=False),
    scratch_types=[
        pltpu.VMEM((CH,), jnp.int32),
        pltpu.VMEM((CH,), jnp.int32),
        pltpu.VMEM((CH,), jnp.int32),
        pltpu.VMEM((CH,), jnp.float32),
        pltpu.VMEM((CNT_PT,), jnp.float32),
        pltpu.VMEM_SHARED((INVSZ,), jnp.float32),
    ],
)(_sc_count_body)


def _zero_rows(rows):
    def _z(r, _):
        for j in range(8):
            rows[r, pl.ds(j * 16, 16)] = jnp.zeros((16,), jnp.float32)
        return 0
    lax.fori_loop(0, CH, _z, 0)


def _zero_acc_stripe(rows, acc, s):
    for k in range(5):
        pltpu.sync_copy(rows, acc.at[pl.ds(s * ROWS_PT + k * CH, CH)])


def _copy_out_stripe(acc, out_hbm, c, s):
    for k in range(5):
        r0 = s * ROWS_PT + k * CH
        pltpu.sync_copy(acc.at[pl.ds(r0, CH)],
                        out_hbm.at[pl.ds(c * ACC_ROWS + r0, CH)])


def _scale_rows(rows, scalesv):
    def _sr(r, _):
        sv = plsc.load_gather(scalesv, [jnp.full((16,), r, jnp.int32)])
        for j in range(8):
            sl = pl.ds(j * 16, 16)
            rows[r, sl] = rows[r, sl] * sv
        return 0
    lax.fori_loop(0, CH, _sr, 0)


def _agg_phase(tab_hbm, src_hbm, dst_hbm, et_hbm, inv_v, acc,
               srcv, dstv, etv, gidxv, scalesv, rows, sem, ebase):
    def body(g, _):
        off = ebase + g * CH
        pltpu.sync_copy(src_hbm.at[pl.ds(off, CH)], srcv)
        pltpu.sync_copy(dst_hbm.at[pl.ds(off, CH)], dstv)
        pltpu.sync_copy(et_hbm.at[pl.ds(off, CH)], etv)
        for j in range(8):
            sl = pl.ds(j * 16, 16)
            tv = etv[sl]
            gidxv[sl] = srcv[sl] * 2 + tv
            civ = dstv[sl] * 2 + tv
            scalesv[sl] = plsc.load_gather(inv_v, [civ])
        cp = pltpu.async_copy(tab_hbm.at[gidxv], rows, sem)
        cp.wait()
        _scale_rows(rows, scalesv)
        pltpu.sync_copy(rows, acc.at[dstv], add=True)
        return 0
    lax.fori_loop(0, NCHUNK, body, 0)


def _enh_phase(x_hbm, src_hbm, dst_hbm, w_hbm, acc,
               srcv, dstv, scalesv, rows, sem, ebase):
    def body(g, _):
        off = ebase + g * CH
        pltpu.sync_copy(src_hbm.at[pl.ds(off, CH)], srcv)
        pltpu.sync_copy(dst_hbm.at[pl.ds(off, CH)], dstv)
        pltpu.sync_copy(w_hbm.at[pl.ds(off, CH)], scalesv)
        for j in range(8):
            sl = pl.ds(j * 16, 16)
            wv = scalesv[sl]
            scalesv[sl] = jnp.where(wv > 0.5, wv * 0.1, 0.0)
        cp = pltpu.async_copy(x_hbm.at[srcv], rows, sem)
        cp.wait()
        _scale_rows(rows, scalesv)
        pltpu.sync_copy(rows, acc.at[dstv], add=True)
        return 0
    lax.fori_loop(0, NCHUNK, body, 0)


def _sc_edge1_real(tab_hbm, x_hbm, inv_hbm, src_hbm, dst_hbm, et_hbm, w_hbm,
                   agg_out, enh_out,
                   srcv, dstv, etv, gidxv, scalesv, rows, inv_v, acc, sem):
    c = lax.axis_index("c")
    s = lax.axis_index("s")
    ebase = c * EPC + s * EPT
    pltpu.sync_copy(inv_hbm, inv_v)
    _zero_rows(rows)
    _zero_acc_stripe(rows, acc, s)
    plsc.subcore_barrier()
    _agg_phase(tab_hbm, src_hbm, dst_hbm, et_hbm, inv_v, acc,
               srcv, dstv, etv, gidxv, scalesv, rows, sem, ebase)
    plsc.subcore_barrier()
    _copy_out_stripe(acc, agg_out, c, s)
    _zero_rows(rows)
    _zero_acc_stripe(rows, acc, s)
    plsc.subcore_barrier()
    _enh_phase(x_hbm, src_hbm, dst_hbm, w_hbm, acc,
               srcv, dstv, scalesv, rows, sem, ebase)
    plsc.subcore_barrier()
    _copy_out_stripe(acc, enh_out, c, s)


_sc_edge1 = functools.partial(
    pl.kernel,
    out_type=(jax.ShapeDtypeStruct((NC * ACC_ROWS, D), jnp.float32),
              jax.ShapeDtypeStruct((NC * ACC_ROWS, D), jnp.float32)),
    mesh=_MESH,
    compiler_params=pltpu.CompilerParams(needs_layout_passes=False),
    scratch_types=[
        pltpu.VMEM((CH,), jnp.int32),
        pltpu.VMEM((CH,), jnp.int32),
        pltpu.VMEM((CH,), jnp.int32),
        pltpu.VMEM((CH,), jnp.int32),
        pltpu.VMEM((CH,), jnp.float32),
        pltpu.VMEM((CH, D), jnp.float32),
        pltpu.VMEM((INVSZ,), jnp.float32),
        pltpu.VMEM_SHARED((ACC_ROWS, D), jnp.float32),
        pltpu.SemaphoreType.DMA,
    ],
)(_sc_edge1_real)


def _sc_edge2_real(tab_hbm, inv_hbm, src_hbm, dst_hbm, et_hbm,
                   agg_out,
                   srcv, dstv, etv, gidxv, scalesv, rows, inv_v, acc, sem):
    c = lax.axis_index("c")
    s = lax.axis_index("s")
    ebase = c * EPC + s * EPT
    pltpu.sync_copy(inv_hbm, inv_v)
    _zero_rows(rows)
    _zero_acc_stripe(rows, acc, s)
    plsc.subcore_barrier()
    _agg_phase(tab_hbm, src_hbm, dst_hbm, et_hbm, inv_v, acc,
               srcv, dstv, etv, gidxv, scalesv, rows, sem, ebase)
    plsc.subcore_barrier()
    _copy_out_stripe(acc, agg_out, c, s)


_sc_edge2 = functools.partial(
    pl.kernel,
    out_type=jax.ShapeDtypeStruct((NC * ACC_ROWS, D), jnp.float32),
    mesh=_MESH,
    compiler_params=pltpu.CompilerParams(needs_layout_passes=False),
    scratch_types=[
        pltpu.VMEM((CH,), jnp.int32),
        pltpu.VMEM((CH,), jnp.int32),
        pltpu.VMEM((CH,), jnp.int32),
        pltpu.VMEM((CH,), jnp.int32),
        pltpu.VMEM((CH,), jnp.float32),
        pltpu.VMEM((CH, D), jnp.float32),
        pltpu.VMEM((INVSZ,), jnp.float32),
        pltpu.VMEM_SHARED((ACC_ROWS, D), jnp.float32),
        pltpu.SemaphoreType.DMA,
    ],
)(_sc_edge2_real)


BLK = 400
GRID = N // BLK


def _tc_inv_body(cnt_ref, out_ref):
    csum = cnt_ref[0] + cnt_ref[1]
    r = lax.broadcasted_iota(jnp.int32, csum.shape, 0)
    col = lax.broadcasted_iota(jnp.int32, csum.shape, 1)
    idx = r * 128 + col
    out_ref[...] = jnp.where(idx < 2 * N, 1.0 / jnp.maximum(csum, 1.0), 0.0)


def _tc_a_body(tw_ref, np_ref, cp_ref, cm_ref,
               W_tw, b_tw, g_tw, be_tw, W_np, b_np, g_np, be_np,
               W_cp, b_cp, g_cp, be_cp, W_cm, b_cm, g_cm, be_cm,
               W_i1, b_i1, g_i1, be_i1, W_i2, b_i2, g_i2, be_i2,
               Wr10, Wr11, Wroot1, brg1,
               x_ref, y_ref, r1_ref):
    f32 = jnp.float32
    t = _lrelu(_bn(jnp.dot(tw_ref[...], W_tw[...], preferred_element_type=f32)
                   + b_tw[...], g_tw[...], be_tw[...]))
    n = _lrelu(_bn(jnp.dot(np_ref[...], W_np[...], preferred_element_type=f32)
                   + b_np[...], g_np[...], be_np[...]))
    c = _lrelu(_bn(jnp.dot(cp_ref[...], W_cp[...], preferred_element_type=f32)
                   + b_cp[...], g_cp[...], be_cp[...]))
    cm = _lrelu(_bn(jnp.dot(cm_ref[...], W_cm[...], preferred_element_type=f32)
                    + b_cm[...], g_cm[...], be_cm[...]))
    x = jnp.concatenate([t, n, c, cm], axis=1)
    x = _lrelu(_bn(jnp.dot(x, W_i1[...], preferred_element_type=f32)
                   + b_i1[...], g_i1[...], be_i1[...]))
    x = _lrelu(_bn(jnp.dot(x, W_i2[...], preferred_element_type=f32)
                   + b_i2[...], g_i2[...], be_i2[...]))
    y0 = jnp.dot(x, Wr10[...], preferred_element_type=f32)
    y1 = jnp.dot(x, Wr11[...], preferred_element_type=f32)
    x_ref[...] = x
    y_ref[...] = jnp.concatenate([y0[:, None, :], y1[:, None, :]], axis=1)
    r1_ref[...] = jnp.dot(x, Wroot1[...], preferred_element_type=f32) + brg1[...]


def _tc_b_body(r1_ref, agg_ref, enh_ref, Wr20, Wr21, Wroot2, brg2,
               g_bn1, be_bn1, z_ref, r2_ref):
    f32 = jnp.float32
    g = r1_ref[...] + agg_ref[0] + agg_ref[1]
    x1 = _bn(g, g_bn1[...], be_bn1[...]) + enh_ref[0] + enh_ref[1]
    z0 = jnp.dot(x1, Wr20[...], preferred_element_type=f32)
    z1 = jnp.dot(x1, Wr21[...], preferred_element_type=f32)
    z_ref[...] = jnp.concatenate([z0[:, None, :], z1[:, None, :]], axis=1)
    r2_ref[...] = jnp.dot(x1, Wroot2[...], preferred_element_type=f32) + brg2[...]


def _tc_c_body(r2_ref, agg_ref, g_bn2, be_bn2,
               W_o1, b_o1, g_o1, be_o1, W_o2, b_o2, out_ref):
    f32 = jnp.float32
    x2 = _bn(r2_ref[...] + agg_ref[0] + agg_ref[1], g_bn2[...], be_bn2[...])
    f = _lrelu(_bn(jnp.dot(x2, W_o1[...], preferred_element_type=f32)
                   + b_o1[...], g_o1[...], be_o1[...]))
    logits = jnp.dot(f, W_o2[...], preferred_element_type=f32) + b_o2[...]
    m = jnp.max(logits, axis=1, keepdims=True)
    lse = m + jnp.log(jnp.sum(jnp.exp(logits - m), axis=1, keepdims=True))
    out_ref[...] = logits - lse


def _row_spec(shape):
    nd = len(shape)
    return pl.BlockSpec((BLK,) + shape[1:],
                        lambda i: (i,) + (0,) * (nd - 1))


def _full_spec(shape):
    nd = len(shape)
    return pl.BlockSpec(shape, lambda i: (0,) * nd)


def _part_spec(shape):
    return pl.BlockSpec((2, BLK, shape[2]), lambda i: (0, i, 0))


def kernel(tweet, num_prop, cat_prop, community_embedding,
           edge_community_weight,
           W_tw, b_tw, g_tw, be_tw, W_np, b_np, g_np, be_np,
           W_cp, b_cp, g_cp, be_cp, W_cm, b_cm, g_cm, be_cm,
           W_i1, b_i1, g_i1, be_i1, W_i2, b_i2, g_i2, be_i2,
           Wrel1, Wroot1, brg1, g_bn1, be_bn1,
           Wrel2, Wroot2, brg2, g_bn2, be_bn2,
           W_o1, b_o1, g_o1, be_o1, W_o2, b_o2,
           edge_index, edge_type):
    f32 = jnp.float32
    i32 = jnp.int32

    npad = E_PAD - E
    src_p = jnp.concatenate([edge_index[0].astype(i32),
                             jnp.zeros((npad,), i32)])
    dst_p = jnp.concatenate([edge_index[1].astype(i32),
                             jnp.full((npad,), N, i32)])
    et_p = jnp.concatenate([edge_type.astype(i32), jnp.full((npad,), 2, i32)])
    w_p = jnp.concatenate([edge_community_weight.astype(f32),
                           jnp.zeros((npad,), f32)])

    vec = lambda v: v.reshape(1, -1)

    cnt = _sc_count(dst_p, et_p)
    inv = pl.pallas_call(
        _tc_inv_body,
        out_shape=jax.ShapeDtypeStruct((INVSZ // 128, 128), f32),
    )(cnt.reshape(NC, INVSZ // 128, 128)).reshape(INVSZ)

    a_ins = [tweet, num_prop, cat_prop, community_embedding,
             W_tw, vec(b_tw), vec(g_tw), vec(be_tw),
             W_np, vec(b_np), vec(g_np), vec(be_np),
             W_cp, vec(b_cp), vec(g_cp), vec(be_cp),
             W_cm, vec(b_cm), vec(g_cm), vec(be_cm),
             W_i1, vec(b_i1), vec(g_i1), vec(be_i1),
             W_i2, vec(b_i2), vec(g_i2), vec(be_i2),
             Wrel1[0], Wrel1[1], Wroot1, vec(brg1)]
    a_specs = ([_row_spec(tweet.shape), _row_spec(num_prop.shape),
                _row_spec(cat_prop.shape), _row_spec(community_embedding.shape)]
               + [_full_spec(a.shape) for a in a_ins[4:]])
    x, yc, r1 = pl.pallas_call(
        _tc_a_body,
        grid=(GRID,),
        in_specs=a_specs,
        out_specs=[_row_spec((N, D)),
                   pl.BlockSpec((BLK, 2, D), lambda i: (i, 0, 0)),
                   _row_spec((N, D))],
        out_shape=[jax.ShapeDtypeStruct((N, D), f32),
                   jax.ShapeDtypeStruct((N, 2, D), f32),
                   jax.ShapeDtypeStruct((N, D), f32)],
    )(*a_ins)

    agg1, enh = _sc_edge1(yc.reshape(2 * N, D), x, inv,
                          src_p, dst_p, et_p, w_p)

    b_ins = [r1, agg1.reshape(2, ACC_ROWS, D), enh.reshape(2, ACC_ROWS, D),
             Wrel2[0], Wrel2[1], Wroot2, vec(brg2), vec(g_bn1), vec(be_bn1)]
    b_specs = [_row_spec((N, D)), _part_spec((2, N, D)), _part_spec((2, N, D)),
               _full_spec((D, D)), _full_spec((D, D)), _full_spec((D, D)),
               _full_spec((1, D)), _full_spec((1, D)), _full_spec((1, D))]
    zc, r2 = pl.pallas_call(
        _tc_b_body,
        grid=(GRID,),
        in_specs=b_specs,
        out_specs=[pl.BlockSpec((BLK, 2, D), lambda i: (i, 0, 0)),
                   _row_spec((N, D))],
        out_shape=[jax.ShapeDtypeStruct((N, 2, D), f32),
                   jax.ShapeDtypeStruct((N, D), f32)],
    )(*b_ins)

    agg2 = _sc_edge2(zc.reshape(2 * N, D), inv, src_p, dst_p, et_p)

    c_ins = [r2, agg2.reshape(2, ACC_ROWS, D), vec(g_bn2), vec(be_bn2),
             W_o1, vec(b_o1), vec(g_o1), vec(be_o1), W_o2, vec(b_o2)]
    c_specs = [_row_spec((N, D)), _part_spec((2, N, D)),
               _full_spec((1, D)), _full_spec((1, D)),
               _full_spec((D, D)), _full_spec((1, D)), _full_spec((1, D)),
               _full_spec((1, D)), _full_spec((D, 2)), _full_spec((1, 2))]
    out = pl.pallas_call(
        _tc_c_body,
        grid=(GRID,),
        in_specs=c_specs,
        out_specs=pl.BlockSpec((BLK, 2), lambda i: (i, 0)),
        out_shape=jax.ShapeDtypeStruct((N, 2), f32),
    )(*c_ins)
    return out


# speedup vs baseline: 3.7432x; 1.1315x over previous
import functools

import jax
import jax.numpy as jnp
import numpy as np
from jax import lax
from jax.experimental import pallas as pl
from jax.experimental.pallas import tpu as pltpu
from jax.experimental.pallas import tpu_sc as plsc

N = 10000
E = 320000
D = 128

NC = 2
NS = 16
CH = 128
SCH = 8
SCHE = SCH * CH
NSUP = 10
NCHUNK = SCH * NSUP
EPT = NCHUNK * CH
EPC = NS * EPT
E_PAD = NC * EPC
ACC_ROWS = 10240
ROWS_PT = ACC_ROWS // NS
INVSZ = 2 * N + 480
CNT_PT = INVSZ // NS

_BN_INV = float(1.0 / np.sqrt(np.float32(1.0 + 1e-5)))

_MESH = plsc.VectorSubcoreMesh(core_axis_name="c", subcore_axis_name="s")

_SC_PARAMS = pltpu.CompilerParams(needs_layout_passes=False)


def _lrelu(x):
    return jnp.where(x >= 0, x, 0.01 * x)


def _bn(x, g, b):
    return x * (g * _BN_INV) + b


def _sc_count_body(cidx_hbm, cnt_out, cidx0, cidx1, onesv, zerov, cacc,
                   isem0, isem1):
    c = lax.axis_index("c")
    s = lax.axis_index("s")
    ebase = c * EPC + s * EPT
    cidxs = (cidx0, cidx1)
    isems = (isem0, isem1)

    def _stage(g, p):
        pltpu.async_copy(cidx_hbm.at[pl.ds(ebase + g * CH, CH)],
                         cidxs[p], isems[p])

    def _wait_stage(g, p):
        pltpu.make_async_copy(cidx_hbm.at[pl.ds(ebase + g * CH, CH)],
                              cidxs[p], isems[p]).wait()

    _stage(0, 0)
    for j in range(8):
        sl = pl.ds(j * 16, 16)
        onesv[sl] = jnp.full((16,), 1.0, jnp.float32)

    def _z(i, _):
        zerov[pl.ds(i * 16, 16)] = jnp.zeros((16,), jnp.float32)
        return 0
    lax.fori_loop(0, CNT_PT // 16, _z, 0)
    pltpu.sync_copy(zerov, cacc.at[pl.ds(s * CNT_PT, CNT_PT)])
    plsc.subcore_barrier()

    def body(h, _):
        for p in range(2):
            g = 2 * h + p
            _wait_stage(g, p)

            @pl.when(g < NCHUNK - 1)
            def _():
                _stage(g + 1, 1 - p)
            pltpu.sync_copy(onesv, cacc.at[cidxs[p]], add=True)
        return 0
    lax.fori_loop(0, NCHUNK // 2, body, 0)
    plsc.subcore_barrier()
    pltpu.sync_copy(cacc.at[pl.ds(s * CNT_PT, CNT_PT)],
                    cnt_out.at[pl.ds(c * INVSZ + s * CNT_PT, CNT_PT)])


_sc_count = functools.partial(
    pl.kernel,
    out_type=jax.ShapeDtypeStruct((NC * INVSZ,), jnp.float32),
    mesh=_MESH,
    compiler_params=_SC_PARAMS,
    scratch_types=[
        pltpu.VMEM((CH,), jnp.int32),
        pltpu.VMEM((CH,), jnp.int32),
        pltpu.VMEM((CH,), jnp.float32),
        pltpu.VMEM((CNT_PT,), jnp.float32),
        pltpu.VMEM_SHARED((INVSZ,), jnp.float32),
        pltpu.SemaphoreType.DMA,
        pltpu.SemaphoreType.DMA,
    ],
)(_sc_count_body)


def _zero_rows(rows):
    def _z(r, _):
        for j in range(8):
            rows[r, pl.ds(j * 16, 16)] = jnp.zeros((16,), jnp.float32)
        return 0
    lax.fori_loop(0, CH, _z, 0)


def _zero_acc_stripe(rows, acc, s):
    for k in range(5):
        pltpu.sync_copy(rows, acc.at[pl.ds(s * ROWS_PT + k * CH, CH)])


def _copy_out_stripe(acc, out_hbm, c, s):
    for k in range(5):
        r0 = s * ROWS_PT + k * CH
        pltpu.sync_copy(acc.at[pl.ds(r0, CH)],
                        out_hbm.at[pl.ds(c * ACC_ROWS + r0, CH)])


def _scale_rows(rows, scalesv):
    def _sr(r, _):
        sv = plsc.load_gather(scalesv, [jnp.full((16,), r, jnp.int32)])
        for j in range(8):
            sl = pl.ds(j * 16, 16)
            rows[r, sl] = rows[r, sl] * sv
        return 0
    lax.fori_loop(0, CH, _sr, 0)


class _EdgeBufs:
    def __init__(self, refs):
        (self.gidxb, self.cidxb, self.wb, self.rows, self.gidxc, self.cidxc,
         self.dsts, self.scl, self.isem, self.gsem, self.sclsem) = refs


def _pipe_phase(tab_hbm, gidx_hbm, cidx_hbm, w_hbm, inv_hbm, acc, B, ebase,
                enh):
    def stage_idx(u, k):
        off = ebase + u * SCHE
        pltpu.async_copy(gidx_hbm.at[pl.ds(off, SCHE)], B.gidxb[k], B.isem[k])
        pltpu.async_copy(cidx_hbm.at[pl.ds(off, SCHE)], B.cidxb[k], B.isem[k])
        if enh:
            pltpu.async_copy(w_hbm.at[pl.ds(off, SCHE)], B.wb[k], B.isem[k])

    def wait_idx(u, k):
        off = ebase + u * SCHE
        pltpu.make_async_copy(gidx_hbm.at[pl.ds(off, SCHE)], B.gidxb[k],
                              B.isem[k]).wait()
        pltpu.make_async_copy(cidx_hbm.at[pl.ds(off, SCHE)], B.cidxb[k],
                              B.isem[k]).wait()
        if enh:
            pltpu.make_async_copy(w_hbm.at[pl.ds(off, SCHE)], B.wb[k],
                                  B.isem[k]).wait()

    def stage_row(q, k, p):
        for j in range(8):
            slb = pl.ds(q * CH + j * 16, 16)
            sl = pl.ds(j * 16, 16)
            cv = B.cidxb[k][slb]
            gv = B.gidxb[k][slb]
            B.dsts[p][sl] = lax.shift_right_logical(cv, 1)
            if enh:
                B.gidxc[p][sl] = lax.shift_right_logical(gv, 1)
                wv = B.wb[k][slb]
                B.scl[p][sl] = jnp.where(wv > 0.5, wv * 0.1, 0.0)
            else:
                B.gidxc[p][sl] = gv
                B.cidxc[p][sl] = cv
        pltpu.async_copy(tab_hbm.at[B.gidxc[p]], B.rows[p], B.gsem[p])
        if not enh:
            pltpu.async_copy(inv_hbm.at[B.cidxc[p]], B.scl[p], B.sclsem[p])

    def wait_row(p):
        pltpu.make_async_copy(tab_hbm.at[B.gidxc[p]], B.rows[p],
                              B.gsem[p]).wait()
        if not enh:
            pltpu.make_async_copy(inv_hbm.at[B.cidxc[p]], B.scl[p],
                                  B.sclsem[p]).wait()

    def process(p):
        _scale_rows(B.rows[p], B.scl[p])
        pltpu.sync_copy(B.rows[p], acc.at[B.dsts[p]], add=True)

    stage_idx(0, 0)
    wait_idx(0, 0)
    stage_row(0, 0, 0)

    def outer(v, _):
        for k in range(2):
            u = 2 * v + k

            @pl.when(u + 1 < NSUP)
            def _():
                stage_idx(u + 1, 1 - k)
            for q in range(SCH):
                p = q % 2
                wait_row(p)
                if q < SCH - 1:
                    stage_row(q + 1, k, 1 - p)
                else:
                    @pl.when(u + 1 < NSUP)
                    def _():
                        wait_idx(u + 1, 1 - k)
                        stage_row(0, 1 - k, 1 - p)
                process(p)
        return 0
    lax.fori_loop(0, NSUP // 2, outer, 0)


_EDGE_SCRATCH = [
    pltpu.VMEM((SCHE,), jnp.int32),
    pltpu.VMEM((SCHE,), jnp.int32),
    pltpu.VMEM((SCHE,), jnp.int32),
    pltpu.VMEM((SCHE,), jnp.int32),
    pltpu.VMEM((CH, D), jnp.float32),
    pltpu.VMEM((CH, D), jnp.float32),
    pltpu.VMEM((CH,), jnp.int32),
    pltpu.VMEM((CH,), jnp.int32),
    pltpu.VMEM((CH,), jnp.int32),
    pltpu.VMEM((CH,), jnp.int32),
    pltpu.VMEM((CH,), jnp.int32),
    pltpu.VMEM((CH,), jnp.int32),
    pltpu.VMEM((CH,), jnp.float32),
    pltpu.VMEM((CH,), jnp.float32),
    pltpu.VMEM_SHARED((ACC_ROWS, D), jnp.float32),
    pltpu.SemaphoreType.DMA,
    pltpu.SemaphoreType.DMA,
    pltpu.SemaphoreType.DMA,
    pltpu.SemaphoreType.DMA,
    pltpu.SemaphoreType.DMA,
    pltpu.SemaphoreType.DMA,
]


def _mk_bufs(gidxb0, gidxb1, cidxb0, cidxb1, rows0, rows1,
             gidxc0, gidxc1, cidxc0, cidxc1, dsts0, dsts1, scl0, scl1,
             isem0, isem1, gsem0, gsem1, sclsem0, sclsem1, wb):
    return _EdgeBufs(((gidxb0, gidxb1), (cidxb0, cidxb1), wb,
                      (rows0, rows1), (gidxc0, gidxc1), (cidxc0, cidxc1),
                      (dsts0, dsts1), (scl0, scl1),
                      (isem0, isem1), (gsem0, gsem1), (sclsem0, sclsem1)))


def _sc_edge1_real(tab_hbm, x_hbm, inv_hbm, gidx_hbm, cidx_hbm, w_hbm,
                   agg_out, enh_out,
                   gidxb0, gidxb1, cidxb0, cidxb1, rows0, rows1,
                   gidxc0, gidxc1, cidxc0, cidxc1, dsts0, dsts1, scl0, scl1,
                   acc, isem0, isem1, gsem0, gsem1, sclsem0, sclsem1,
                   wb0, wb1):
    c = lax.axis_index("c")
    s = lax.axis_index("s")
    ebase = c * EPC + s * EPT
    B = _mk_bufs(gidxb0, gidxb1, cidxb0, cidxb1, rows0, rows1,
                 gidxc0, gidxc1, cidxc0, cidxc1, dsts0, dsts1, scl0, scl1,
                 isem0, isem1, gsem0, gsem1, sclsem0, sclsem1, (wb0, wb1))
    _zero_rows(rows0)
    _zero_acc_stripe(rows0, acc, s)
    plsc.subcore_barrier()
    _pipe_phase(tab_hbm, gidx_hbm, cidx_hbm, w_hbm, inv_hbm, acc, B, ebase,
                enh=False)
    plsc.subcore_barrier()
    _copy_out_stripe(acc, agg_out, c, s)
    _zero_rows(rows0)
    _zero_acc_stripe(rows0, acc, s)
    plsc.subcore_barrier()
    _pipe_phase(x_hbm, gidx_hbm, cidx_hbm, w_hbm, inv_hbm, acc, B, ebase,
                enh=True)
    plsc.subcore_barrier()
    _copy_out_stripe(acc, enh_out, c, s)


_sc_edge1 = functools.partial(
    pl.kernel,
    out_type=(jax.ShapeDtypeStruct((NC * ACC_ROWS, D), jnp.float32),
              jax.ShapeDtypeStruct((NC * ACC_ROWS, D), jnp.float32)),
    mesh=_MESH,
    compiler_params=_SC_PARAMS,
    scratch_types=_EDGE_SCRATCH + [pltpu.VMEM((SCHE,), jnp.float32),
                                   pltpu.VMEM((SCHE,), jnp.float32)],
)(_sc_edge1_real)


def _sc_edge2_real(tab_hbm, inv_hbm, gidx_hbm, cidx_hbm,
                   agg_out,
                   gidxb0, gidxb1, cidxb0, cidxb1, rows0, rows1,
                   gidxc0, gidxc1, cidxc0, cidxc1, dsts0, dsts1, scl0, scl1,
                   acc, isem0, isem1, gsem0, gsem1, sclsem0, sclsem1):
    c = lax.axis_index("c")
    s = lax.axis_index("s")
    ebase = c * EPC + s * EPT
    B = _mk_bufs(gidxb0, gidxb1, cidxb0, cidxb1, rows0, rows1,
                 gidxc0, gidxc1, cidxc0, cidxc1, dsts0, dsts1, scl0, scl1,
                 isem0, isem1, gsem0, gsem1, sclsem0, sclsem1, None)
    _zero_rows(rows0)
    _zero_acc_stripe(rows0, acc, s)
    plsc.subcore_barrier()
    _pipe_phase(tab_hbm, gidx_hbm, cidx_hbm, None, inv_hbm, acc, B, ebase,
                enh=False)
    plsc.subcore_barrier()
    _copy_out_stripe(acc, agg_out, c, s)


_sc_edge2 = functools.partial(
    pl.kernel,
    out_type=jax.ShapeDtypeStruct((NC * ACC_ROWS, D), jnp.float32),
    mesh=_MESH,
    compiler_params=_SC_PARAMS,
    scratch_types=_EDGE_SCRATCH,
)(_sc_edge2_real)


BLK = 400
GRID = N // BLK


def _tc_inv_body(cnt_ref, out_ref):
    csum = cnt_ref[0] + cnt_ref[1]
    r = lax.broadcasted_iota(jnp.int32, csum.shape, 0)
    col = lax.broadcasted_iota(jnp.int32, csum.shape, 1)
    idx = r * 128 + col
    out_ref[...] = jnp.where(idx < 2 * N, 1.0 / jnp.maximum(csum, 1.0), 0.0)


def _tc_a_body(tw_ref, np_ref, cp_ref, cm_ref,
               W_tw, b_tw, g_tw, be_tw, W_np, b_np, g_np, be_np,
               W_cp, b_cp, g_cp, be_cp, W_cm, b_cm, g_cm, be_cm,
               W_i1, b_i1, g_i1, be_i1, W_i2, b_i2, g_i2, be_i2,
               Wr10, Wr11, Wroot1, brg1,
               x_ref, y_ref, r1_ref):
    f32 = jnp.float32
    t = _lrelu(_bn(jnp.dot(tw_ref[...], W_tw[...], preferred_element_type=f32)
                   + b_tw[...], g_tw[...], be_tw[...]))
    n = _lrelu(_bn(jnp.dot(np_ref[...], W_np[...], preferred_element_type=f32)
                   + b_np[...], g_np[...], be_np[...]))
    c = _lrelu(_bn(jnp.dot(cp_ref[...], W_cp[...], preferred_element_type=f32)
                   + b_cp[...], g_cp[...], be_cp[...]))
    cm = _lrelu(_bn(jnp.dot(cm_ref[...], W_cm[...], preferred_element_type=f32)
                    + b_cm[...], g_cm[...], be_cm[...]))
    x = jnp.concatenate([t, n, c, cm], axis=1)
    x = _lrelu(_bn(jnp.dot(x, W_i1[...], preferred_element_type=f32)
                   + b_i1[...], g_i1[...], be_i1[...]))
    x = _lrelu(_bn(jnp.dot(x, W_i2[...], preferred_element_type=f32)
                   + b_i2[...], g_i2[...], be_i2[...]))
    y0 = jnp.dot(x, Wr10[...], preferred_element_type=f32)
    y1 = jnp.dot(x, Wr11[...], preferred_element_type=f32)
    x_ref[...] = x
    y_ref[...] = jnp.concatenate([y0[:, None, :], y1[:, None, :]], axis=1)
    r1_ref[...] = jnp.dot(x, Wroot1[...], preferred_element_type=f32) + brg1[...]


def _tc_b_body(r1_ref, agg_ref, enh_ref, Wr20, Wr21, Wroot2, brg2,
               g_bn1, be_bn1, z_ref, r2_ref):
    f32 = jnp.float32
    g = r1_ref[...] + agg_ref[0] + agg_ref[1]
    x1 = _bn(g, g_bn1[...], be_bn1[...]) + enh_ref[0] + enh_ref[1]
    z0 = jnp.dot(x1, Wr20[...], preferred_element_type=f32)
    z1 = jnp.dot(x1, Wr21[...], preferred_element_type=f32)
    z_ref[...] = jnp.concatenate([z0[:, None, :], z1[:, None, :]], axis=1)
    r2_ref[...] = jnp.dot(x1, Wroot2[...], preferred_element_type=f32) + brg2[...]


def _tc_c_body(r2_ref, agg_ref, g_bn2, be_bn2,
               W_o1, b_o1, g_o1, be_o1, W_o2, b_o2, out_ref):
    f32 = jnp.float32
    x2 = _bn(r2_ref[...] + agg_ref[0] + agg_ref[1], g_bn2[...], be_bn2[...])
    f = _lrelu(_bn(jnp.dot(x2, W_o1[...], preferred_element_type=f32)
                   + b_o1[...], g_o1[...], be_o1[...]))
    logits = jnp.dot(f, W_o2[...], preferred_element_type=f32) + b_o2[...]
    m = jnp.max(logits, axis=1, keepdims=True)
    lse = m + jnp.log(jnp.sum(jnp.exp(logits - m), axis=1, keepdims=True))
    out_ref[...] = logits - lse


def _row_spec(shape):
    nd = len(shape)
    return pl.BlockSpec((BLK,) + shape[1:],
                        lambda i: (i,) + (0,) * (nd - 1))


def _full_spec(shape):
    nd = len(shape)
    return pl.BlockSpec(shape, lambda i: (0,) * nd)


def _part_spec(shape):
    return pl.BlockSpec((2, BLK, shape[2]), lambda i: (0, i, 0))


def kernel(tweet, num_prop, cat_prop, community_embedding,
           edge_community_weight,
           W_tw, b_tw, g_tw, be_tw, W_np, b_np, g_np, be_np,
           W_cp, b_cp, g_cp, be_cp, W_cm, b_cm, g_cm, be_cm,
           W_i1, b_i1, g_i1, be_i1, W_i2, b_i2, g_i2, be_i2,
           Wrel1, Wroot1, brg1, g_bn1, be_bn1,
           Wrel2, Wroot2, brg2, g_bn2, be_bn2,
           W_o1, b_o1, g_o1, be_o1, W_o2, b_o2,
           edge_index, edge_type):
    f32 = jnp.float32
    i32 = jnp.int32

    npad = E_PAD - E
    src = edge_index[0].astype(i32)
    dst = edge_index[1].astype(i32)
    et = edge_type.astype(i32)
    gidx_p = jnp.concatenate([src * 2 + et, jnp.full((npad,), 2, i32)])
    cidx_p = jnp.concatenate([dst * 2 + et, jnp.full((npad,), 2 * N + 2, i32)])
    w_p = jnp.concatenate([edge_community_weight.astype(f32),
                           jnp.zeros((npad,), f32)])

    vec = lambda v: v.reshape(1, -1)

    cnt = _sc_count(cidx_p)
    inv = pl.pallas_call(
        _tc_inv_body,
        out_shape=jax.ShapeDtypeStruct((INVSZ // 128, 128), f32),
    )(cnt.reshape(NC, INVSZ // 128, 128)).reshape(INVSZ)

    a_ins = [tweet, num_prop, cat_prop, community_embedding,
             W_tw, vec(b_tw), vec(g_tw), vec(be_tw),
             W_np, vec(b_np), vec(g_np), vec(be_np),
             W_cp, vec(b_cp), vec(g_cp), vec(be_cp),
             W_cm, vec(b_cm), vec(g_cm), vec(be_cm),
             W_i1, vec(b_i1), vec(g_i1), vec(be_i1),
             W_i2, vec(b_i2), vec(g_i2), vec(be_i2),
             Wrel1[0], Wrel1[1], Wroot1, vec(brg1)]
    a_specs = ([_row_spec(tweet.shape), _row_spec(num_prop.shape),
                _row_spec(cat_prop.shape), _row_spec(community_embedding.shape)]
               + [_full_spec(a.shape) for a in a_ins[4:]])
    x, yc, r1 = pl.pallas_call(
        _tc_a_body,
        grid=(GRID,),
        in_specs=a_specs,
        out_specs=[_row_spec((N, D)),
                   pl.BlockSpec((BLK, 2, D), lambda i: (i, 0, 0)),
                   _row_spec((N, D))],
        out_shape=[jax.ShapeDtypeStruct((N, D), f32),
                   jax.ShapeDtypeStruct((N, 2, D), f32),
                   jax.ShapeDtypeStruct((N, D), f32)],
    )(*a_ins)

    agg1, enh = _sc_edge1(yc.reshape(2 * N, D), x, inv, gidx_p, cidx_p, w_p)

    b_ins = [r1, agg1.reshape(2, ACC_ROWS, D), enh.reshape(2, ACC_ROWS, D),
             Wrel2[0], Wrel2[1], Wroot2, vec(brg2), vec(g_bn1), vec(be_bn1)]
    b_specs = [_row_spec((N, D)), _part_spec((2, N, D)), _part_spec((2, N, D)),
               _full_spec((D, D)), _full_spec((D, D)), _full_spec((D, D)),
               _full_spec((1, D)), _full_spec((1, D)), _full_spec((1, D))]
    zc, r2 = pl.pallas_call(
        _tc_b_body,
        grid=(GRID,),
        in_specs=b_specs,
        out_specs=[pl.BlockSpec((BLK, 2, D), lambda i: (i, 0, 0)),
                   _row_spec((N, D))],
        out_shape=[jax.ShapeDtypeStruct((N, 2, D), f32),
                   jax.ShapeDtypeStruct((N, D), f32)],
    )(*b_ins)

    agg2 = _sc_edge2(zc.reshape(2 * N, D), inv, gidx_p, cidx_p)

    c_ins = [r2, agg2.reshape(2, ACC_ROWS, D), vec(g_bn2), vec(be_bn2),
             W_o1, vec(b_o1), vec(g_o1), vec(be_o1), W_o2, vec(b_o2)]
    c_specs = [_row_spec((N, D)), _part_spec((2, N, D)),
               _full_spec((1, D)), _full_spec((1, D)),
               _full_spec((D, D)), _full_spec((1, D)), _full_spec((1, D)),
               _full_spec((1, D)), _full_spec((D, 2)), _full_spec((1, 2))]
    out = pl.pallas_call(
        _tc_c_body,
        grid=(GRID,),
        in_specs=c_specs,
        out_specs=pl.BlockSpec((BLK, 2), lambda i: (i, 0)),
        out_shape=jax.ShapeDtypeStruct((N, 2), f32),
    )(*c_ins)
    return out


# speedup vs baseline: 3.7482x; 1.0013x over previous
import functools

import jax
import jax.numpy as jnp
import numpy as np
from jax import lax
from jax.experimental import pallas as pl
from jax.experimental.pallas import tpu as pltpu
from jax.experimental.pallas import tpu_sc as plsc

N = 10000
E = 320000
D = 128

NC = 2
NS = 16
CH = 128
SCH = 8
SCHE = SCH * CH
NSUP = 10
NCHUNK = SCH * NSUP
EPT = NCHUNK * CH
EPC = NS * EPT
E_PAD = NC * EPC
ACC_ROWS = 10240
ROWS_PT = ACC_ROWS // NS
INVSZ = 2 * N + 480
CNT_PT = INVSZ // NS

_BN_INV = float(1.0 / np.sqrt(np.float32(1.0 + 1e-5)))

_MESH = plsc.VectorSubcoreMesh(core_axis_name="c", subcore_axis_name="s")

_SC_PARAMS = pltpu.CompilerParams(needs_layout_passes=False)


def _lrelu(x):
    return jnp.where(x >= 0, x, 0.01 * x)


def _bn(x, g, b):
    return x * (g * _BN_INV) + b


def _sc_count_body(cidx_hbm, cnt_out, cidx0, cidx1, onesv, zerov, cacc,
                   isem0, isem1):
    c = lax.axis_index("c")
    s = lax.axis_index("s")
    ebase = c * EPC + s * EPT
    cidxs = (cidx0, cidx1)
    isems = (isem0, isem1)

    def _stage(g, p):
        pltpu.async_copy(cidx_hbm.at[pl.ds(ebase + g * CH, CH)],
                         cidxs[p], isems[p])

    def _wait_stage(g, p):
        pltpu.make_async_copy(cidx_hbm.at[pl.ds(ebase + g * CH, CH)],
                              cidxs[p], isems[p]).wait()

    _stage(0, 0)
    for j in range(8):
        sl = pl.ds(j * 16, 16)
        onesv[sl] = jnp.full((16,), 1.0, jnp.float32)

    def _z(i, _):
        zerov[pl.ds(i * 16, 16)] = jnp.zeros((16,), jnp.float32)
        return 0
    lax.fori_loop(0, CNT_PT // 16, _z, 0)
    pltpu.sync_copy(zerov, cacc.at[pl.ds(s * CNT_PT, CNT_PT)])
    plsc.subcore_barrier()

    def body(h, _):
        for p in range(2):
            g = 2 * h + p
            _wait_stage(g, p)

            @pl.when(g < NCHUNK - 1)
            def _():
                _stage(g + 1, 1 - p)
            pltpu.sync_copy(onesv, cacc.at[cidxs[p]], add=True)
        return 0
    lax.fori_loop(0, NCHUNK // 2, body, 0)
    plsc.subcore_barrier()
    pltpu.sync_copy(cacc.at[pl.ds(s * CNT_PT, CNT_PT)],
                    cnt_out.at[pl.ds(c * INVSZ + s * CNT_PT, CNT_PT)])


_sc_count = functools.partial(
    pl.kernel,
    out_type=jax.ShapeDtypeStruct((NC * INVSZ,), jnp.float32),
    mesh=_MESH,
    compiler_params=_SC_PARAMS,
    scratch_types=[
        pltpu.VMEM((CH,), jnp.int32),
        pltpu.VMEM((CH,), jnp.int32),
        pltpu.VMEM((CH,), jnp.float32),
        pltpu.VMEM((CNT_PT,), jnp.float32),
        pltpu.VMEM_SHARED((INVSZ,), jnp.float32),
        pltpu.SemaphoreType.DMA,
        pltpu.SemaphoreType.DMA,
    ],
)(_sc_count_body)


def _zero_rows(rows):
    def _z(r, _):
        for j in range(8):
            rows[r, pl.ds(j * 16, 16)] = jnp.zeros((16,), jnp.float32)
        return 0
    lax.fori_loop(0, CH, _z, 0)


def _zero_acc_stripe(rows, acc, s, sem):
    cps = [pltpu.async_copy(rows, acc.at[pl.ds(s * ROWS_PT + k * CH, CH)], sem)
           for k in range(5)]
    for cp in cps:
        cp.wait()


def _copy_out_stripe(acc, out_hbm, c, s, sem):
    cps = []
    for k in range(5):
        r0 = s * ROWS_PT + k * CH
        cps.append(pltpu.async_copy(acc.at[pl.ds(r0, CH)],
                                    out_hbm.at[pl.ds(c * ACC_ROWS + r0, CH)],
                                    sem))
    for cp in cps:
        cp.wait()


def _scale_rows(rows, scalesv):
    def _sr(i, _):
        for rr in range(4):
            r = i * 4 + rr
            sv = plsc.load_gather(scalesv, [jnp.full((16,), r, jnp.int32)])
            for j in range(8):
                sl = pl.ds(j * 16, 16)
                rows[r, sl] = rows[r, sl] * sv
        return 0
    lax.fori_loop(0, CH // 4, _sr, 0)


class _EdgeBufs:
    def __init__(self, refs):
        (self.gidxb, self.cidxb, self.wb, self.rows, self.gidxc, self.cidxc,
         self.dsts, self.scl, self.isem, self.gsem, self.sclsem,
         self.ssem) = refs


def _pipe_phase(tab_hbm, gidx_hbm, cidx_hbm, w_hbm, inv_hbm, acc, B, ebase,
                enh):
    def stage_idx(u, k):
        off = ebase + u * SCHE
        pltpu.async_copy(gidx_hbm.at[pl.ds(off, SCHE)], B.gidxb[k], B.isem[k])
        pltpu.async_copy(cidx_hbm.at[pl.ds(off, SCHE)], B.cidxb[k], B.isem[k])
        if enh:
            pltpu.async_copy(w_hbm.at[pl.ds(off, SCHE)], B.wb[k], B.isem[k])

    def wait_idx(u, k):
        off = ebase + u * SCHE
        pltpu.make_async_copy(gidx_hbm.at[pl.ds(off, SCHE)], B.gidxb[k],
                              B.isem[k]).wait()
        pltpu.make_async_copy(cidx_hbm.at[pl.ds(off, SCHE)], B.cidxb[k],
                              B.isem[k]).wait()
        if enh:
            pltpu.make_async_copy(w_hbm.at[pl.ds(off, SCHE)], B.wb[k],
                                  B.isem[k]).wait()

    def stage_row(q, k, p):
        for j in range(8):
            slb = pl.ds(q * CH + j * 16, 16)
            sl = pl.ds(j * 16, 16)
            cv = B.cidxb[k][slb]
            gv = B.gidxb[k][slb]
            B.dsts[p][sl] = lax.shift_right_logical(cv, 1)
            if enh:
                B.gidxc[p][sl] = lax.shift_right_logical(gv, 1)
                wv = B.wb[k][slb]
                B.scl[p][sl] = jnp.where(wv > 0.5, wv * 0.1, 0.0)
            else:
                B.gidxc[p][sl] = gv
                B.cidxc[p][sl] = cv
        pltpu.async_copy(tab_hbm.at[B.gidxc[p]], B.rows[p], B.gsem[p])
        if not enh:
            pltpu.async_copy(inv_hbm.at[B.cidxc[p]], B.scl[p], B.sclsem[p])

    def wait_row(p):
        pltpu.make_async_copy(tab_hbm.at[B.gidxc[p]], B.rows[p],
                              B.gsem[p]).wait()
        if not enh:
            pltpu.make_async_copy(inv_hbm.at[B.cidxc[p]], B.scl[p],
                                  B.sclsem[p]).wait()

    def process(p):
        _scale_rows(B.rows[p], B.scl[p])
        pltpu.async_copy(B.rows[p], acc.at[B.dsts[p]], B.ssem[p], add=True)

    def wait_scatter(p):
        pltpu.make_async_copy(B.rows[p], acc.at[B.dsts[p]], B.ssem[p]).wait()

    stage_idx(0, 0)
    wait_idx(0, 0)
    stage_row(0, 0, 0)

    def outer(v, _):
        for k in range(2):
            u = 2 * v + k

            @pl.when(u + 1 < NSUP)
            def _():
                stage_idx(u + 1, 1 - k)
            for q in range(SCH):
                p = q % 2
                wait_row(p)
                if q < SCH - 1:
                    if q == 0:
                        @pl.when(u >= 1)
                        def _():
                            wait_scatter(1 - p)
                    else:
                        wait_scatter(1 - p)
                    stage_row(q + 1, k, 1 - p)
                else:
                    @pl.when(u + 1 < NSUP)
                    def _():
                        wait_idx(u + 1, 1 - k)
                        wait_scatter(1 - p)
                        stage_row(0, 1 - k, 1 - p)
                process(p)
        return 0
    lax.fori_loop(0, NSUP // 2, outer, 0)
    wait_scatter(0)
    wait_scatter(1)


_EDGE_SCRATCH = [
    pltpu.VMEM((SCHE,), jnp.int32),
    pltpu.VMEM((SCHE,), jnp.int32),
    pltpu.VMEM((SCHE,), jnp.int32),
    pltpu.VMEM((SCHE,), jnp.int32),
    pltpu.VMEM((CH, D), jnp.float32),
    pltpu.VMEM((CH, D), jnp.float32),
    pltpu.VMEM((CH,), jnp.int32),
    pltpu.VMEM((CH,), jnp.int32),
    pltpu.VMEM((CH,), jnp.int32),
    pltpu.VMEM((CH,), jnp.int32),
    pltpu.VMEM((CH,), jnp.int32),
    pltpu.VMEM((CH,), jnp.int32),
    pltpu.VMEM((CH,), jnp.float32),
    pltpu.VMEM((CH,), jnp.float32),
    pltpu.VMEM_SHARED((ACC_ROWS, D), jnp.float32),
    pltpu.SemaphoreType.DMA,
    pltpu.SemaphoreType.DMA,
    pltpu.SemaphoreType.DMA,
    pltpu.SemaphoreType.DMA,
    pltpu.SemaphoreType.DMA,
    pltpu.SemaphoreType.DMA,
    pltpu.SemaphoreType.DMA,
    pltpu.SemaphoreType.DMA,
]


def _mk_bufs(gidxb0, gidxb1, cidxb0, cidxb1, rows0, rows1,
             gidxc0, gidxc1, cidxc0, cidxc1, dsts0, dsts1, scl0, scl1,
             isem0, isem1, gsem0, gsem1, sclsem0, sclsem1, ssem0, ssem1, wb):
    return _EdgeBufs(((gidxb0, gidxb1), (cidxb0, cidxb1), wb,
                      (rows0, rows1), (gidxc0, gidxc1), (cidxc0, cidxc1),
                      (dsts0, dsts1), (scl0, scl1),
                      (isem0, isem1), (gsem0, gsem1), (sclsem0, sclsem1),
                      (ssem0, ssem1)))


def _sc_edge1_real(tab_hbm, x_hbm, inv_hbm, gidx_hbm, cidx_hbm, w_hbm,
                   agg_out, enh_out,
                   gidxb0, gidxb1, cidxb0, cidxb1, rows0, rows1,
                   gidxc0, gidxc1, cidxc0, cidxc1, dsts0, dsts1, scl0, scl1,
                   acc, isem0, isem1, gsem0, gsem1, sclsem0, sclsem1,
                   ssem0, ssem1, wb0, wb1):
    c = lax.axis_index("c")
    s = lax.axis_index("s")
    ebase = c * EPC + s * EPT
    B = _mk_bufs(gidxb0, gidxb1, cidxb0, cidxb1, rows0, rows1,
                 gidxc0, gidxc1, cidxc0, cidxc1, dsts0, dsts1, scl0, scl1,
                 isem0, isem1, gsem0, gsem1, sclsem0, sclsem1,
                 ssem0, ssem1, (wb0, wb1))
    _zero_rows(rows0)
    _zero_acc_stripe(rows0, acc, s, isem0)
    plsc.subcore_barrier()
    _pipe_phase(tab_hbm, gidx_hbm, cidx_hbm, w_hbm, inv_hbm, acc, B, ebase,
                enh=False)
    plsc.subcore_barrier()
    _copy_out_stripe(acc, agg_out, c, s, isem0)
    _zero_rows(rows0)
    _zero_acc_stripe(rows0, acc, s, isem0)
    plsc.subcore_barrier()
    _pipe_phase(x_hbm, gidx_hbm, cidx_hbm, w_hbm, inv_hbm, acc, B, ebase,
                enh=True)
    plsc.subcore_barrier()
    _copy_out_stripe(acc, enh_out, c, s, isem0)


_sc_edge1 = functools.partial(
    pl.kernel,
    out_type=(jax.ShapeDtypeStruct((NC * ACC_ROWS, D), jnp.float32),
              jax.ShapeDtypeStruct((NC * ACC_ROWS, D), jnp.float32)),
    mesh=_MESH,
    compiler_params=_SC_PARAMS,
    scratch_types=_EDGE_SCRATCH + [pltpu.VMEM((SCHE,), jnp.float32),
                                   pltpu.VMEM((SCHE,), jnp.float32)],
)(_sc_edge1_real)


def _sc_edge2_real(tab_hbm, inv_hbm, gidx_hbm, cidx_hbm,
                   agg_out,
                   gidxb0, gidxb1, cidxb0, cidxb1, rows0, rows1,
                   gidxc0, gidxc1, cidxc0, cidxc1, dsts0, dsts1, scl0, scl1,
                   acc, isem0, isem1, gsem0, gsem1, sclsem0, sclsem1,
                   ssem0, ssem1):
    c = lax.axis_index("c")
    s = lax.axis_index("s")
    ebase = c * EPC + s * EPT
    B = _mk_bufs(gidxb0, gidxb1, cidxb0, cidxb1, rows0, rows1,
                 gidxc0, gidxc1, cidxc0, cidxc1, dsts0, dsts1, scl0, scl1,
                 isem0, isem1, gsem0, gsem1, sclsem0, sclsem1,
                 ssem0, ssem1, None)
    _zero_rows(rows0)
    _zero_acc_stripe(rows0, acc, s, isem0)
    plsc.subcore_barrier()
    _pipe_phase(tab_hbm, gidx_hbm, cidx_hbm, None, inv_hbm, acc, B, ebase,
                enh=False)
    plsc.subcore_barrier()
    _copy_out_stripe(acc, agg_out, c, s, isem0)


_sc_edge2 = functools.partial(
    pl.kernel,
    out_type=jax.ShapeDtypeStruct((NC * ACC_ROWS, D), jnp.float32),
    mesh=_MESH,
    compiler_params=_SC_PARAMS,
    scratch_types=_EDGE_SCRATCH,
)(_sc_edge2_real)


BLK = 400
GRID = N // BLK


def _tc_inv_body(cnt_ref, out_ref):
    csum = cnt_ref[0] + cnt_ref[1]
    r = lax.broadcasted_iota(jnp.int32, csum.shape, 0)
    col = lax.broadcasted_iota(jnp.int32, csum.shape, 1)
    idx = r * 128 + col
    out_ref[...] = jnp.where(idx < 2 * N, 1.0 / jnp.maximum(csum, 1.0), 0.0)


def _tc_a_body(tw_ref, np_ref, cp_ref, cm_ref,
               W_tw, b_tw, g_tw, be_tw, W_np, b_np, g_np, be_np,
               W_cp, b_cp, g_cp, be_cp, W_cm, b_cm, g_cm, be_cm,
               W_i1, b_i1, g_i1, be_i1, W_i2, b_i2, g_i2, be_i2,
               Wr10, Wr11, Wroot1, brg1,
               x_ref, y_ref, r1_ref):
    f32 = jnp.float32
    t = _lrelu(_bn(jnp.dot(tw_ref[...], W_tw[...], preferred_element_type=f32)
                   + b_tw[...], g_tw[...], be_tw[...]))
    n = _lrelu(_bn(jnp.dot(np_ref[...], W_np[...], preferred_element_type=f32)
                   + b_np[...], g_np[...], be_np[...]))
    c = _lrelu(_bn(jnp.dot(cp_ref[...], W_cp[...], preferred_element_type=f32)
                   + b_cp[...], g_cp[...], be_cp[...]))
    cm = _lrelu(_bn(jnp.dot(cm_ref[...], W_cm[...], preferred_element_type=f32)
                    + b_cm[...], g_cm[...], be_cm[...]))
    x = jnp.concatenate([t, n, c, cm], axis=1)
    x = _lrelu(_bn(jnp.dot(x, W_i1[...], preferred_element_type=f32)
                   + b_i1[...], g_i1[...], be_i1[...]))
    x = _lrelu(_bn(jnp.dot(x, W_i2[...], preferred_element_type=f32)
                   + b_i2[...], g_i2[...], be_i2[...]))
    y0 = jnp.dot(x, Wr10[...], preferred_element_type=f32)
    y1 = jnp.dot(x, Wr11[...], preferred_element_type=f32)
    x_ref[...] = x
    y_ref[...] = jnp.concatenate([y0[:, None, :], y1[:, None, :]], axis=1)
    r1_ref[...] = jnp.dot(x, Wroot1[...], preferred_element_type=f32) + brg1[...]


def _tc_b_body(r1_ref, agg_ref, enh_ref, Wr20, Wr21, Wroot2, brg2,
               g_bn1, be_bn1, z_ref, r2_ref):
    f32 = jnp.float32
    g = r1_ref[...] + agg_ref[0] + agg_ref[1]
    x1 = _bn(g, g_bn1[...], be_bn1[...]) + enh_ref[0] + enh_ref[1]
    z0 = jnp.dot(x1, Wr20[...], preferred_element_type=f32)
    z1 = jnp.dot(x1, Wr21[...], preferred_element_type=f32)
    z_ref[...] = jnp.concatenate([z0[:, None, :], z1[:, None, :]], axis=1)
    r2_ref[...] = jnp.dot(x1, Wroot2[...], preferred_element_type=f32) + brg2[...]


def _tc_c_body(r2_ref, agg_ref, g_bn2, be_bn2,
               W_o1, b_o1, g_o1, be_o1, W_o2, b_o2, out_ref):
    f32 = jnp.float32
    x2 = _bn(r2_ref[...] + agg_ref[0] + agg_ref[1], g_bn2[...], be_bn2[...])
    f = _lrelu(_bn(jnp.dot(x2, W_o1[...], preferred_element_type=f32)
                   + b_o1[...], g_o1[...], be_o1[...]))
    logits = jnp.dot(f, W_o2[...], preferred_element_type=f32) + b_o2[...]
    m = jnp.max(logits, axis=1, keepdims=True)
    lse = m + jnp.log(jnp.sum(jnp.exp(logits - m), axis=1, keepdims=True))
    out_ref[...] = logits - lse


def _row_spec(shape):
    nd = len(shape)
    return pl.BlockSpec((BLK,) + shape[1:],
                        lambda i: (i,) + (0,) * (nd - 1))


def _full_spec(shape):
    nd = len(shape)
    return pl.BlockSpec(shape, lambda i: (0,) * nd)


def _part_spec(shape):
    return pl.BlockSpec((2, BLK, shape[2]), lambda i: (0, i, 0))


def kernel(tweet, num_prop, cat_prop, community_embedding,
           edge_community_weight,
           W_tw, b_tw, g_tw, be_tw, W_np, b_np, g_np, be_np,
           W_cp, b_cp, g_cp, be_cp, W_cm, b_cm, g_cm, be_cm,
           W_i1, b_i1, g_i1, be_i1, W_i2, b_i2, g_i2, be_i2,
           Wrel1, Wroot1, brg1, g_bn1, be_bn1,
           Wrel2, Wroot2, brg2, g_bn2, be_bn2,
           W_o1, b_o1, g_o1, be_o1, W_o2, b_o2,
           edge_index, edge_type):
    f32 = jnp.float32
    i32 = jnp.int32

    npad = E_PAD - E
    src = edge_index[0].astype(i32)
    dst = edge_index[1].astype(i32)
    et = edge_type.astype(i32)
    gidx_p = jnp.concatenate([src * 2 + et, jnp.full((npad,), 2, i32)])
    cidx_p = jnp.concatenate([dst * 2 + et, jnp.full((npad,), 2 * N + 2, i32)])
    w_p = jnp.concatenate([edge_community_weight.astype(f32),
                           jnp.zeros((npad,), f32)])

    vec = lambda v: v.reshape(1, -1)

    cnt = _sc_count(cidx_p)
    inv = pl.pallas_call(
        _tc_inv_body,
        out_shape=jax.ShapeDtypeStruct((INVSZ // 128, 128), f32),
    )(cnt.reshape(NC, INVSZ // 128, 128)).reshape(INVSZ)

    a_ins = [tweet, num_prop, cat_prop, community_embedding,
             W_tw, vec(b_tw), vec(g_tw), vec(be_tw),
             W_np, vec(b_np), vec(g_np), vec(be_np),
             W_cp, vec(b_cp), vec(g_cp), vec(be_cp),
             W_cm, vec(b_cm), vec(g_cm), vec(be_cm),
             W_i1, vec(b_i1), vec(g_i1), vec(be_i1),
             W_i2, vec(b_i2), vec(g_i2), vec(be_i2),
             Wrel1[0], Wrel1[1], Wroot1, vec(brg1)]
    a_specs = ([_row_spec(tweet.shape), _row_spec(num_prop.shape),
                _row_spec(cat_prop.shape), _row_spec(community_embedding.shape)]
               + [_full_spec(a.shape) for a in a_ins[4:]])
    x, yc, r1 = pl.pallas_call(
        _tc_a_body,
        grid=(GRID,),
        in_specs=a_specs,
        out_specs=[_row_spec((N, D)),
                   pl.BlockSpec((BLK, 2, D), lambda i: (i, 0, 0)),
                   _row_spec((N, D))],
        out_shape=[jax.ShapeDtypeStruct((N, D), f32),
                   jax.ShapeDtypeStruct((N, 2, D), f32),
                   jax.ShapeDtypeStruct((N, D), f32)],
    )(*a_ins)

    agg1, enh = _sc_edge1(yc.reshape(2 * N, D), x, inv, gidx_p, cidx_p, w_p)

    b_ins = [r1, agg1.reshape(2, ACC_ROWS, D), enh.reshape(2, ACC_ROWS, D),
             Wrel2[0], Wrel2[1], Wroot2, vec(brg2), vec(g_bn1), vec(be_bn1)]
    b_specs = [_row_spec((N, D)), _part_spec((2, N, D)), _part_spec((2, N, D)),
               _full_spec((D, D)), _full_spec((D, D)), _full_spec((D, D)),
               _full_spec((1, D)), _full_spec((1, D)), _full_spec((1, D))]
    zc, r2 = pl.pallas_call(
        _tc_b_body,
        grid=(GRID,),
        in_specs=b_specs,
        out_specs=[pl.BlockSpec((BLK, 2, D), lambda i: (i, 0, 0)),
                   _row_spec((N, D))],
        out_shape=[jax.ShapeDtypeStruct((N, 2, D), f32),
                   jax.ShapeDtypeStruct((N, D), f32)],
    )(*b_ins)

    agg2 = _sc_edge2(zc.reshape(2 * N, D), inv, gidx_p, cidx_p)

    c_ins = [r2, agg2.reshape(2, ACC_ROWS, D), vec(g_bn2), vec(be_bn2),
             W_o1, vec(b_o1), vec(g_o1), vec(be_o1), W_o2, vec(b_o2)]
    c_specs = [_row_spec((N, D)), _part_spec((2, N, D)),
               _full_spec((1, D)), _full_spec((1, D)),
               _full_spec((D, D)), _full_spec((1, D)), _full_spec((1, D)),
               _full_spec((1, D)), _full_spec((D, 2)), _full_spec((1, 2))]
    out = pl.pallas_call(
        _tc_c_body,
        grid=(GRID,),
        in_specs=c_specs,
        out_specs=pl.BlockSpec((BLK, 2), lambda i: (i, 0)),
        out_shape=jax.ShapeDtypeStruct((N, 2), f32),
    )(*c_ins)
    return out


# speedup vs baseline: 4.3551x; 1.1619x over previous
import functools

import jax
import jax.numpy as jnp
import numpy as np
from jax import lax
from jax.experimental import pallas as pl
from jax.experimental.pallas import tpu as pltpu
from jax.experimental.pallas import tpu_sc as plsc

N = 10000
E = 320000
D = 128

NC = 2
NS = 16
CH = 128
SCH = 8
SCHE = SCH * CH
NSUP = 10
NCHUNK = SCH * NSUP
EPT = NCHUNK * CH
EPC = NS * EPT
E_PAD = NC * EPC
ACC_ROWS = 10240
ROWS_PT = ACC_ROWS // NS
INVSZ = 2 * N + 480
CNT_PT = INVSZ // NS

CH1 = 32
SCH1 = 16
SCHE1 = SCH1 * CH1
NSUP1 = EPT // SCHE1

_BN_INV = float(1.0 / np.sqrt(np.float32(1.0 + 1e-5)))

_MESH = plsc.VectorSubcoreMesh(core_axis_name="c", subcore_axis_name="s")

_SC_PARAMS = pltpu.CompilerParams(needs_layout_passes=False)


def _lrelu(x):
    return jnp.where(x >= 0, x, 0.01 * x)


def _bn(x, g, b):
    return x * (g * _BN_INV) + b


def _sc_count_body(cidx_hbm, cnt_out, cidx0, cidx1, onesv, zerov, cacc,
                   isem0, isem1):
    c = lax.axis_index("c")
    s = lax.axis_index("s")
    ebase = c * EPC + s * EPT
    cidxs = (cidx0, cidx1)
    isems = (isem0, isem1)

    def _stage(g, p):
        pltpu.async_copy(cidx_hbm.at[pl.ds(ebase + g * CH, CH)],
                         cidxs[p], isems[p])

    def _wait_stage(g, p):
        pltpu.make_async_copy(cidx_hbm.at[pl.ds(ebase + g * CH, CH)],
                              cidxs[p], isems[p]).wait()

    _stage(0, 0)
    for j in range(8):
        sl = pl.ds(j * 16, 16)
        onesv[sl] = jnp.full((16,), 1.0, jnp.float32)

    def _z(i, _):
        zerov[pl.ds(i * 16, 16)] = jnp.zeros((16,), jnp.float32)
        return 0
    lax.fori_loop(0, CNT_PT // 16, _z, 0)
    pltpu.sync_copy(zerov, cacc.at[pl.ds(s * CNT_PT, CNT_PT)])
    plsc.subcore_barrier()

    def body(h, _):
        for p in range(2):
            g = 2 * h + p
            _wait_stage(g, p)

            @pl.when(g < NCHUNK - 1)
            def _():
                _stage(g + 1, 1 - p)
            pltpu.sync_copy(onesv, cacc.at[cidxs[p]], add=True)
        return 0
    lax.fori_loop(0, NCHUNK // 2, body, 0)
    plsc.subcore_barrier()
    pltpu.sync_copy(cacc.at[pl.ds(s * CNT_PT, CNT_PT)],
                    cnt_out.at[pl.ds(c * INVSZ + s * CNT_PT, CNT_PT)])


_sc_count = functools.partial(
    pl.kernel,
    out_type=jax.ShapeDtypeStruct((NC * INVSZ,), jnp.float32),
    mesh=_MESH,
    compiler_params=_SC_PARAMS,
    scratch_types=[
        pltpu.VMEM((CH,), jnp.int32),
        pltpu.VMEM((CH,), jnp.int32),
        pltpu.VMEM((CH,), jnp.float32),
        pltpu.VMEM((CNT_PT,), jnp.float32),
        pltpu.VMEM_SHARED((INVSZ,), jnp.float32),
        pltpu.SemaphoreType.DMA,
        pltpu.SemaphoreType.DMA,
    ],
)(_sc_count_body)


def _zero_rows(rows):
    def _z(r, _):
        for j in range(8):
            rows[r, pl.ds(j * 16, 16)] = jnp.zeros((16,), jnp.float32)
        return 0
    lax.fori_loop(0, CH, _z, 0)


def _zero_acc_stripe(rows, acc, s, sem):
    cps = [pltpu.async_copy(rows, acc.at[pl.ds(s * ROWS_PT + k * CH, CH)], sem)
           for k in range(5)]
    for cp in cps:
        cp.wait()


def _copy_out_stripe(acc, out_hbm, c, s, sem):
    cps = []
    for k in range(5):
        r0 = s * ROWS_PT + k * CH
        cps.append(pltpu.async_copy(acc.at[pl.ds(r0, CH)],
                                    out_hbm.at[pl.ds(c * ACC_ROWS + r0, CH)],
                                    sem))
    for cp in cps:
        cp.wait()


def _scale_rows(rows, scalesv):
    def _sr(i, _):
        for rr in range(4):
            r = i * 4 + rr
            sv = plsc.load_gather(scalesv, [jnp.full((16,), r, jnp.int32)])
            for j in range(8):
                sl = pl.ds(j * 16, 16)
                rows[r, sl] = rows[r, sl] * sv
        return 0
    lax.fori_loop(0, CH // 4, _sr, 0)


class _EdgeBufs:
    def __init__(self, refs):
        (self.gidxb, self.cidxb, self.wb, self.rows, self.gidxc, self.cidxc,
         self.dsts, self.scl, self.isem, self.gsem, self.sclsem,
         self.ssem) = refs


def _pipe_phase(tab_hbm, gidx_hbm, cidx_hbm, w_hbm, inv_hbm, acc, B, ebase,
                enh):
    def stage_idx(u, k):
        off = ebase + u * SCHE
        pltpu.async_copy(gidx_hbm.at[pl.ds(off, SCHE)], B.gidxb[k], B.isem[k])
        pltpu.async_copy(cidx_hbm.at[pl.ds(off, SCHE)], B.cidxb[k], B.isem[k])
        if enh:
            pltpu.async_copy(w_hbm.at[pl.ds(off, SCHE)], B.wb[k], B.isem[k])

    def wait_idx(u, k):
        off = ebase + u * SCHE
        pltpu.make_async_copy(gidx_hbm.at[pl.ds(off, SCHE)], B.gidxb[k],
                              B.isem[k]).wait()
        pltpu.make_async_copy(cidx_hbm.at[pl.ds(off, SCHE)], B.cidxb[k],
                              B.isem[k]).wait()
        if enh:
            pltpu.make_async_copy(w_hbm.at[pl.ds(off, SCHE)], B.wb[k],
                                  B.isem[k]).wait()

    def stage_row(q, k, p):
        for j in range(8):
            slb = pl.ds(q * CH + j * 16, 16)
            sl = pl.ds(j * 16, 16)
            cv = B.cidxb[k][slb]
            gv = B.gidxb[k][slb]
            B.dsts[p][sl] = lax.shift_right_logical(cv, 1)
            if enh:
                B.gidxc[p][sl] = lax.shift_right_logical(gv, 1)
                wv = B.wb[k][slb]
                B.scl[p][sl] = jnp.where(wv > 0.5, wv * 0.1, 0.0)
            else:
                B.gidxc[p][sl] = gv
                B.cidxc[p][sl] = cv
        pltpu.async_copy(tab_hbm.at[B.gidxc[p]], B.rows[p], B.gsem[p])
        if not enh:
            pltpu.async_copy(inv_hbm.at[B.cidxc[p]], B.scl[p], B.sclsem[p])

    def wait_row(p):
        pltpu.make_async_copy(tab_hbm.at[B.gidxc[p]], B.rows[p],
                              B.gsem[p]).wait()
        if not enh:
            pltpu.make_async_copy(inv_hbm.at[B.cidxc[p]], B.scl[p],
                                  B.sclsem[p]).wait()

    def process(p):
        _scale_rows(B.rows[p], B.scl[p])
        pltpu.async_copy(B.rows[p], acc.at[B.dsts[p]], B.ssem[p], add=True)

    def wait_scatter(p):
        pltpu.make_async_copy(B.rows[p], acc.at[B.dsts[p]], B.ssem[p]).wait()

    stage_idx(0, 0)
    wait_idx(0, 0)
    stage_row(0, 0, 0)

    def outer(v, _):
        for k in range(2):
            u = 2 * v + k

            @pl.when(u + 1 < NSUP)
            def _():
                stage_idx(u + 1, 1 - k)
            for q in range(SCH):
                p = q % 2
                wait_row(p)
                if q < SCH - 1:
                    if q == 0:
                        @pl.when(u >= 1)
                        def _():
                            wait_scatter(1 - p)
                    else:
                        wait_scatter(1 - p)
                    stage_row(q + 1, k, 1 - p)
                else:
                    @pl.when(u + 1 < NSUP)
                    def _():
                        wait_idx(u + 1, 1 - k)
                        wait_scatter(1 - p)
                        stage_row(0, 1 - k, 1 - p)
                process(p)
        return 0
    lax.fori_loop(0, NSUP // 2, outer, 0)
    wait_scatter(0)
    wait_scatter(1)


_EDGE_SCRATCH = [
    pltpu.VMEM((SCHE,), jnp.int32),
    pltpu.VMEM((SCHE,), jnp.int32),
    pltpu.VMEM((SCHE,), jnp.int32),
    pltpu.VMEM((SCHE,), jnp.int32),
    pltpu.VMEM((CH, D), jnp.float32),
    pltpu.VMEM((CH, D), jnp.float32),
    pltpu.VMEM((CH,), jnp.int32),
    pltpu.VMEM((CH,), jnp.int32),
    pltpu.VMEM((CH,), jnp.int32),
    pltpu.VMEM((CH,), jnp.int32),
    pltpu.VMEM((CH,), jnp.int32),
    pltpu.VMEM((CH,), jnp.int32),
    pltpu.VMEM((CH,), jnp.float32),
    pltpu.VMEM((CH,), jnp.float32),
    pltpu.VMEM_SHARED((ACC_ROWS, D), jnp.float32),
    pltpu.SemaphoreType.DMA,
    pltpu.SemaphoreType.DMA,
    pltpu.SemaphoreType.DMA,
    pltpu.SemaphoreType.DMA,
    pltpu.SemaphoreType.DMA,
    pltpu.SemaphoreType.DMA,
    pltpu.SemaphoreType.DMA,
    pltpu.SemaphoreType.DMA,
]


def _mk_bufs(gidxb0, gidxb1, cidxb0, cidxb1, rows0, rows1,
             gidxc0, gidxc1, cidxc0, cidxc1, dsts0, dsts1, scl0, scl1,
             isem0, isem1, gsem0, gsem1, sclsem0, sclsem1, ssem0, ssem1, wb):
    return _EdgeBufs(((gidxb0, gidxb1), (cidxb0, cidxb1), wb,
                      (rows0, rows1), (gidxc0, gidxc1), (cidxc0, cidxc1),
                      (dsts0, dsts1), (scl0, scl1),
                      (isem0, isem1), (gsem0, gsem1), (sclsem0, sclsem1),
                      (ssem0, ssem1)))


def _pipe_combined(tab_hbm, gidx_hbm, cidx_hbm, w_hbm, inv_hbm, acc,
                   gidxb, cidxb, wb, rows, outb, gidxc, cidxc, dsts,
                   scli, sclw, isem, gsem, sclsem, ssem, ebase):
    def stage_idx(u, k):
        off = ebase + u * SCHE1
        pltpu.async_copy(gidx_hbm.at[pl.ds(off, SCHE1)], gidxb[k], isem[k])
        pltpu.async_copy(cidx_hbm.at[pl.ds(off, SCHE1)], cidxb[k], isem[k])
        pltpu.async_copy(w_hbm.at[pl.ds(off, SCHE1)], wb[k], isem[k])

    def wait_idx(u, k):
        off = ebase + u * SCHE1
        pltpu.make_async_copy(gidx_hbm.at[pl.ds(off, SCHE1)], gidxb[k],
                              isem[k]).wait()
        pltpu.make_async_copy(cidx_hbm.at[pl.ds(off, SCHE1)], cidxb[k],
                              isem[k]).wait()
        pltpu.make_async_copy(w_hbm.at[pl.ds(off, SCHE1)], wb[k],
                              isem[k]).wait()

    def stage_row(q, k, p):
        for j in range(CH1 // 16):
            slb = pl.ds(q * CH1 + j * 16, 16)
            sl = pl.ds(j * 16, 16)
            cv = cidxb[k][slb]
            gidxc[p][sl] = gidxb[k][slb]
            cidxc[p][sl] = cv
            dsts[p][sl] = lax.shift_right_logical(cv, 1)
            wv = wb[k][slb]
            sclw[p][sl] = jnp.where(wv > 0.5, wv, 0.0)
        pltpu.async_copy(tab_hbm.at[gidxc[p]], rows[p], gsem[p])
        pltpu.async_copy(inv_hbm.at[cidxc[p]], scli[p], sclsem[p])

    def wait_row(p):
        pltpu.make_async_copy(tab_hbm.at[gidxc[p]], rows[p], gsem[p]).wait()
        pltpu.make_async_copy(inv_hbm.at[cidxc[p]], scli[p],
                              sclsem[p]).wait()

    def process(p):
        def _fr(i, _):
            for rr in range(4):
                r = i * 4 + rr
                ridx = jnp.full((16,), r, jnp.int32)
                svi = plsc.load_gather(scli[p], [ridx])
                svw = plsc.load_gather(sclw[p], [ridx])
                for j in range(8):
                    sl = pl.ds(j * 16, 16)
                    sr = pl.ds(D + j * 16, 16)
                    outb[p][r, sl] = (rows[p][r, sl] * svi
                                      + rows[p][r, sr] * svw)
            return 0
        lax.fori_loop(0, CH1 // 4, _fr, 0)
        pltpu.async_copy(outb[p], acc.at[dsts[p]], ssem[p], add=True)

    def wait_scatter(p):
        pltpu.make_async_copy(outb[p], acc.at[dsts[p]], ssem[p]).wait()

    stage_idx(0, 0)
    wait_idx(0, 0)
    stage_row(0, 0, 0)

    def outer(v, _):
        for k in range(2):
            u = 2 * v + k

            @pl.when(u + 1 < NSUP1)
            def _():
                stage_idx(u + 1, 1 - k)
            for q in range(SCH1):
                p = q % 2
                wait_row(p)
                if q < SCH1 - 1:
                    if q == 0:
                        @pl.when(u >= 1)
                        def _():
                            wait_scatter(1 - p)
                    else:
                        wait_scatter(1 - p)
                    stage_row(q + 1, k, 1 - p)
                else:
                    @pl.when(u + 1 < NSUP1)
                    def _():
                        wait_idx(u + 1, 1 - k)
                        wait_scatter(1 - p)
                        stage_row(0, 1 - k, 1 - p)
                process(p)
        return 0
    lax.fori_loop(0, NSUP1 // 2, outer, 0)
    wait_scatter(0)
    wait_scatter(1)


def _zero_outb(outb):
    def _z(r, _):
        for j in range(8):
            outb[r, pl.ds(j * 16, 16)] = jnp.zeros((16,), jnp.float32)
        return 0
    lax.fori_loop(0, CH1, _z, 0)


def _sc_edge1c_real(tab_hbm, inv_hbm, gidx_hbm, cidx_hbm, w_hbm,
                    agg_out,
                    gidxb0, gidxb1, cidxb0, cidxb1, wb0, wb1,
                    rows0, rows1, outb0, outb1,
                    gidxc0, gidxc1, cidxc0, cidxc1, dsts0, dsts1,
                    scli0, scli1, sclw0, sclw1,
                    acc, isem0, isem1, gsem0, gsem1, sclsem0, sclsem1,
                    ssem0, ssem1):
    c = lax.axis_index("c")
    s = lax.axis_index("s")
    ebase = c * EPC + s * EPT
    _zero_outb(outb0)
    cps = [pltpu.async_copy(outb0,
                            acc.at[pl.ds(s * ROWS_PT + k * CH1, CH1)], isem0)
           for k in range(ROWS_PT // CH1)]
    for cp in cps:
        cp.wait()
    plsc.subcore_barrier()
    _pipe_combined(tab_hbm, gidx_hbm, cidx_hbm, w_hbm, inv_hbm, acc,
                   (gidxb0, gidxb1), (cidxb0, cidxb1), (wb0, wb1),
                   (rows0, rows1), (outb0, outb1),
                   (gidxc0, gidxc1), (cidxc0, cidxc1), (dsts0, dsts1),
                   (scli0, scli1), (sclw0, sclw1),
                   (isem0, isem1), (gsem0, gsem1), (sclsem0, sclsem1),
                   (ssem0, ssem1), ebase)
    plsc.subcore_barrier()
    _copy_out_stripe(acc, agg_out, c, s, isem0)


_sc_edge1c = functools.partial(
    pl.kernel,
    out_type=jax.ShapeDtypeStruct((NC * ACC_ROWS, D), jnp.float32),
    mesh=_MESH,
    compiler_params=_SC_PARAMS,
    scratch_types=[
        pltpu.VMEM((SCHE1,), jnp.int32),
        pltpu.VMEM((SCHE1,), jnp.int32),
        pltpu.VMEM((SCHE1,), jnp.int32),
        pltpu.VMEM((SCHE1,), jnp.int32),
        pltpu.VMEM((SCHE1,), jnp.float32),
        pltpu.VMEM((SCHE1,), jnp.float32),
        pltpu.VMEM((CH1, 2 * D), jnp.float32),
        pltpu.VMEM((CH1, 2 * D), jnp.float32),
        pltpu.VMEM((CH1, D), jnp.float32),
        pltpu.VMEM((CH1, D), jnp.float32),
        pltpu.VMEM((CH1,), jnp.int32),
        pltpu.VMEM((CH1,), jnp.int32),
        pltpu.VMEM((CH1,), jnp.int32),
        pltpu.VMEM((CH1,), jnp.int32),
        pltpu.VMEM((CH1,), jnp.int32),
        pltpu.VMEM((CH1,), jnp.int32),
        pltpu.VMEM((CH1,), jnp.float32),
        pltpu.VMEM((CH1,), jnp.float32),
        pltpu.VMEM((CH1,), jnp.float32),
        pltpu.VMEM((CH1,), jnp.float32),
        pltpu.VMEM_SHARED((ACC_ROWS, D), jnp.float32),
        pltpu.SemaphoreType.DMA,
        pltpu.SemaphoreType.DMA,
        pltpu.SemaphoreType.DMA,
        pltpu.SemaphoreType.DMA,
        pltpu.SemaphoreType.DMA,
        pltpu.SemaphoreType.DMA,
        pltpu.SemaphoreType.DMA,
        pltpu.SemaphoreType.DMA,
    ],
)(_sc_edge1c_real)


def _sc_edge2_real(tab_hbm, inv_hbm, gidx_hbm, cidx_hbm,
                   agg_out,
                   gidxb0, gidxb1, cidxb0, cidxb1, rows0, rows1,
                   gidxc0, gidxc1, cidxc0, cidxc1, dsts0, dsts1, scl0, scl1,
                   acc, isem0, isem1, gsem0, gsem1, sclsem0, sclsem1,
                   ssem0, ssem1):
    c = lax.axis_index("c")
    s = lax.axis_index("s")
    ebase = c * EPC + s * EPT
    B = _mk_bufs(gidxb0, gidxb1, cidxb0, cidxb1, rows0, rows1,
                 gidxc0, gidxc1, cidxc0, cidxc1, dsts0, dsts1, scl0, scl1,
                 isem0, isem1, gsem0, gsem1, sclsem0, sclsem1,
                 ssem0, ssem1, None)
    _zero_rows(rows0)
    _zero_acc_stripe(rows0, acc, s, isem0)
    plsc.subcore_barrier()
    _pipe_phase(tab_hbm, gidx_hbm, cidx_hbm, None, inv_hbm, acc, B, ebase,
                enh=False)
    plsc.subcore_barrier()
    _copy_out_stripe(acc, agg_out, c, s, isem0)


_sc_edge2 = functools.partial(
    pl.kernel,
    out_type=jax.ShapeDtypeStruct((NC * ACC_ROWS, D), jnp.float32),
    mesh=_MESH,
    compiler_params=_SC_PARAMS,
    scratch_types=_EDGE_SCRATCH,
)(_sc_edge2_real)


BLK = 400
GRID = N // BLK


def _tc_inv_body(cnt_ref, out_ref):
    csum = cnt_ref[0] + cnt_ref[1]
    r = lax.broadcasted_iota(jnp.int32, csum.shape, 0)
    col = lax.broadcasted_iota(jnp.int32, csum.shape, 1)
    idx = r * 128 + col
    out_ref[...] = jnp.where(idx < 2 * N, 1.0 / jnp.maximum(csum, 1.0), 0.0)


def _tc_a_body(tw_ref, np_ref, cp_ref, cm_ref,
               W_tw, b_tw, g_tw, be_tw, W_np, b_np, g_np, be_np,
               W_cp, b_cp, g_cp, be_cp, W_cm, b_cm, g_cm, be_cm,
               W_i1, b_i1, g_i1, be_i1, W_i2, b_i2, g_i2, be_i2,
               Wr10, Wr11, Wroot1, brg1, g_bn1,
               y_ref, r1_ref):
    f32 = jnp.float32
    t = _lrelu(_bn(jnp.dot(tw_ref[...], W_tw[...], preferred_element_type=f32)
                   + b_tw[...], g_tw[...], be_tw[...]))
    n = _lrelu(_bn(jnp.dot(np_ref[...], W_np[...], preferred_element_type=f32)
                   + b_np[...], g_np[...], be_np[...]))
    c = _lrelu(_bn(jnp.dot(cp_ref[...], W_cp[...], preferred_element_type=f32)
                   + b_cp[...], g_cp[...], be_cp[...]))
    cm = _lrelu(_bn(jnp.dot(cm_ref[...], W_cm[...], preferred_element_type=f32)
                    + b_cm[...], g_cm[...], be_cm[...]))
    x = jnp.concatenate([t, n, c, cm], axis=1)
    x = _lrelu(_bn(jnp.dot(x, W_i1[...], preferred_element_type=f32)
                   + b_i1[...], g_i1[...], be_i1[...]))
    x = _lrelu(_bn(jnp.dot(x, W_i2[...], preferred_element_type=f32)
                   + b_i2[...], g_i2[...], be_i2[...]))
    y0 = jnp.dot(x, Wr10[...], preferred_element_type=f32)
    y1 = jnp.dot(x, Wr11[...], preferred_element_type=f32)
    xs = x * (0.1 / (g_bn1[...] * _BN_INV))
    y_ref[...] = jnp.concatenate(
        [jnp.concatenate([y0, xs], axis=1)[:, None, :],
         jnp.concatenate([y1, xs], axis=1)[:, None, :]], axis=1)
    r1_ref[...] = jnp.dot(x, Wroot1[...], preferred_element_type=f32) + brg1[...]


def _tc_b_body(r1_ref, agg_ref, Wr20, Wr21, Wroot2, brg2,
               g_bn1, be_bn1, z_ref, r2_ref):
    f32 = jnp.float32
    g = r1_ref[...] + agg_ref[0] + agg_ref[1]
    x1 = _bn(g, g_bn1[...], be_bn1[...])
    z0 = jnp.dot(x1, Wr20[...], preferred_element_type=f32)
    z1 = jnp.dot(x1, Wr21[...], preferred_element_type=f32)
    z_ref[...] = jnp.concatenate([z0[:, None, :], z1[:, None, :]], axis=1)
    r2_ref[...] = jnp.dot(x1, Wroot2[...], preferred_element_type=f32) + brg2[...]


def _tc_c_body(r2_ref, agg_ref, g_bn2, be_bn2,
               W_o1, b_o1, g_o1, be_o1, W_o2, b_o2, out_ref):
    f32 = jnp.float32
    x2 = _bn(r2_ref[...] + agg_ref[0] + agg_ref[1], g_bn2[...], be_bn2[...])
    f = _lrelu(_bn(jnp.dot(x2, W_o1[...], preferred_element_type=f32)
                   + b_o1[...], g_o1[...], be_o1[...]))
    logits = jnp.dot(f, W_o2[...], preferred_element_type=f32) + b_o2[...]
    m = jnp.max(logits, axis=1, keepdims=True)
    lse = m + jnp.log(jnp.sum(jnp.exp(logits - m), axis=1, keepdims=True))
    out_ref[...] = logits - lse


def _row_spec(shape):
    nd = len(shape)
    return pl.BlockSpec((BLK,) + shape[1:],
                        lambda i: (i,) + (0,) * (nd - 1))


def _full_spec(shape):
    nd = len(shape)
    return pl.BlockSpec(shape, lambda i: (0,) * nd)


def _part_spec(shape):
    return pl.BlockSpec((2, BLK, shape[2]), lambda i: (0, i, 0))


def kernel(tweet, num_prop, cat_prop, community_embedding,
           edge_community_weight,
           W_tw, b_tw, g_tw, be_tw, W_np, b_np, g_np, be_np,
           W_cp, b_cp, g_cp, be_cp, W_cm, b_cm, g_cm, be_cm,
           W_i1, b_i1, g_i1, be_i1, W_i2, b_i2, g_i2, be_i2,
           Wrel1, Wroot1, brg1, g_bn1, be_bn1,
           Wrel2, Wroot2, brg2, g_bn2, be_bn2,
           W_o1, b_o1, g_o1, be_o1, W_o2, b_o2,
           edge_index, edge_type):
    f32 = jnp.float32
    i32 = jnp.int32

    npad = E_PAD - E
    src = edge_index[0].astype(i32)
    dst = edge_index[1].astype(i32)
    et = edge_type.astype(i32)
    gidx_p = jnp.concatenate([src * 2 + et, jnp.full((npad,), 2, i32)])
    cidx_p = jnp.concatenate([dst * 2 + et, jnp.full((npad,), 2 * N + 2, i32)])
    w_p = jnp.concatenate([edge_community_weight.astype(f32),
                           jnp.zeros((npad,), f32)])

    vec = lambda v: v.reshape(1, -1)

    cnt = _sc_count(cidx_p)
    inv = pl.pallas_call(
        _tc_inv_body,
        out_shape=jax.ShapeDtypeStruct((INVSZ // 128, 128), f32),
    )(cnt.reshape(NC, INVSZ // 128, 128)).reshape(INVSZ)

    a_ins = [tweet, num_prop, cat_prop, community_embedding,
             W_tw, vec(b_tw), vec(g_tw), vec(be_tw),
             W_np, vec(b_np), vec(g_np), vec(be_np),
             W_cp, vec(b_cp), vec(g_cp), vec(be_cp),
             W_cm, vec(b_cm), vec(g_cm), vec(be_cm),
             W_i1, vec(b_i1), vec(g_i1), vec(be_i1),
             W_i2, vec(b_i2), vec(g_i2), vec(be_i2),
             Wrel1[0], Wrel1[1], Wroot1, vec(brg1), vec(g_bn1)]
    a_specs = ([_row_spec(tweet.shape), _row_spec(num_prop.shape),
                _row_spec(cat_prop.shape), _row_spec(community_embedding.shape)]
               + [_full_spec(a.shape) for a in a_ins[4:]])
    yc, r1 = pl.pallas_call(
        _tc_a_body,
        grid=(GRID,),
        in_specs=a_specs,
        out_specs=[pl.BlockSpec((BLK, 2, 2 * D), lambda i: (i, 0, 0)),
                   _row_spec((N, D))],
        out_shape=[jax.ShapeDtypeStruct((N, 2, 2 * D), f32),
                   jax.ShapeDtypeStruct((N, D), f32)],
    )(*a_ins)

    agg1 = _sc_edge1c(yc.reshape(2 * N, 2 * D), inv, gidx_p, cidx_p, w_p)

    b_ins = [r1, agg1.reshape(2, ACC_ROWS, D),
             Wrel2[0], Wrel2[1], Wroot2, vec(brg2), vec(g_bn1), vec(be_bn1)]
    b_specs = [_row_spec((N, D)), _part_spec((2, N, D)),
               _full_spec((D, D)), _full_spec((D, D)), _full_spec((D, D)),
               _full_spec((1, D)), _full_spec((1, D)), _full_spec((1, D))]
    zc, r2 = pl.pallas_call(
        _tc_b_body,
        grid=(GRID,),
        in_specs=b_specs,
        out_specs=[pl.BlockSpec((BLK, 2, D), lambda i: (i, 0, 0)),
                   _row_spec((N, D))],
        out_shape=[jax.ShapeDtypeStruct((N, 2, D), f32),
                   jax.ShapeDtypeStruct((N, D), f32)],
    )(*b_ins)

    agg2 = _sc_edge2(zc.reshape(2 * N, D), inv, gidx_p, cidx_p)

    c_ins = [r2, agg2.reshape(2, ACC_ROWS, D), vec(g_bn2), vec(be_bn2),
             W_o1, vec(b_o1), vec(g_o1), vec(be_o1), W_o2, vec(b_o2)]
    c_specs = [_row_spec((N, D)), _part_spec((2, N, D)),
               _full_spec((1, D)), _full_spec((1, D)),
               _full_spec((D, D)), _full_spec((1, D)), _full_spec((1, D)),
               _full_spec((1, D)), _full_spec((D, 2)), _full_spec((1, 2))]
    out = pl.pallas_call(
        _tc_c_body,
        grid=(GRID,),
        in_specs=c_specs,
        out_specs=pl.BlockSpec((BLK, 2), lambda i: (i, 0)),
        out_shape=jax.ShapeDtypeStruct((N, 2), f32),
    )(*c_ins)
    return out
